# baseline (speedup 1.0000x reference)
import jax
import jax.numpy as jnp
from jax.experimental import pallas as pl
from jax.experimental.pallas import tpu as pltpu
from jax.experimental.pallas import tpu_sc as plsc

F = 128
NCORES = 2
NSUB = 16
CPW = 2 * F // (NCORES * NSUB)
LANES = 16


def kernel(mask, row_embed, col_embed):
    b, h, w = mask.shape
    mesh = plsc.VectorSubcoreMesh(core_axis_name="c", subcore_axis_name="s")

    @pl.kernel(
        out_type=jax.ShapeDtypeStruct((b, 2 * F, h, w), jnp.float32),
        mesh=mesh,
        scratch_types=[
            pltpu.VMEM((CPW, h, w), jnp.float32),
            pltpu.VMEM((CPW, h), jnp.float32),
            pltpu.VMEM_SHARED((h, F), jnp.float32),
            pltpu.SemaphoreType.DMA,
        ],
    )
    def sc_kernel(row_hbm, col_hbm, out_hbm, chunk, tab, tabS, sem):
        core = jax.lax.axis_index("c")
        sub = jax.lax.axis_index("s")
        c0 = core * F + sub * CPW

        @pl.when(jnp.logical_and(core == 0, sub == 0))
        def _stage_col():
            pltpu.async_copy(col_hbm.at[pl.ds(0, w)], tabS, sem).wait()

        @pl.when(jnp.logical_and(core == 1, sub == 0))
        def _stage_row():
            pltpu.async_copy(row_hbm.at[pl.ds(0, h)], tabS, sem).wait()

        plsc.subcore_barrier()

        for j in range(CPW):
            pltpu.async_copy(
                tabS.at[pl.ds(0, h), sub * CPW + j], tab.at[j], sem
            ).wait()

        @pl.when(core == 0)
        def _x_half():
            for j in range(CPW):
                for xh in range(w // LANES):
                    v = tab.at[j][pl.ds(xh * LANES, LANES)]
                    for y in range(h):
                        chunk.at[j].at[y][pl.ds(xh * LANES, LANES)] = v

        @pl.when(core == 1)
        def _y_half():
            for j in range(CPW):
                for yh in range(h // LANES):
                    hv = tab.at[j][pl.ds(yh * LANES, LANES)]
                    for yl in range(LANES):
                        y = yh * LANES + yl
                        v = jnp.zeros((LANES,), jnp.float32) + hv[yl]
                        for xh in range(w // LANES):
                            chunk.at[j].at[y][pl.ds(xh * LANES, LANES)] = v

        for bi in range(b):
            pltpu.sync_copy(chunk, out_hbm.at[bi, pl.ds(c0, CPW)])

    return sc_kernel(row_embed, col_embed)
